# baseline (speedup 1.0000x reference)
import functools
import math

import jax
import jax.numpy as jnp
from jax import lax
from jax.experimental import pallas as pl
from jax.experimental.pallas import tpu as pltpu
from jax.experimental.pallas import tpu_sc as plsc

N = 10000
E = 320000
D = 128
D2 = 64
NCLS = 10
NCLS_PAD = 16

NC = 2
NS = 16
NW = NC * NS
EPW = E // NW

CH_A = 40
NCH_A = EPW // CH_A
CH_E = 80
NCH_E = EPW // CH_E

ROWS_PER_TILE = N // NS

_MESH = plsc.VectorSubcoreMesh(core_axis_name="c", subcore_axis_name="s")


def _ring4(nchunk, gissue, consume, cdrain):
  gissue(0, 0)
  gissue(1, 1)

  @pl.loop(0, nchunk // 4)
  def _(j):
    for p in range(4):
      c = 4 * j + p
      b2 = (p + 2) % 4
      if p >= 2:
        cdrain(b2)
      else:
        pl.when(c >= 2)(functools.partial(cdrain, b2))
      pl.when(c + 2 < nchunk)(functools.partial(gissue, c + 2, b2))
      consume(c, p)

  rem = nchunk % 4
  for t in range(rem):
    consume(nchunk - rem + t, t)
  for c in range(nchunk - rem - 2, nchunk):
    cdrain(c % 4)


def _agg_body(h_hbm, z0_hbm, src_hbm, dst_hbm, out0_hbm, out1_hbm,
              idx_s, idx_d, rows0, rows1, rows2, rows3, agg,
              sg0, sg1, sg2, sg3, ss0, ss1, ss2, ss3):
  cid = lax.axis_index("c")
  sid = lax.axis_index("s")
  wid = sid * NC + cid

  rsl = pl.ds(sid * ROWS_PER_TILE, ROWS_PER_TILE)

  @pl.when(cid == 0)
  def _():
    pltpu.sync_copy(h_hbm.at[rsl], agg.at[rsl])

  @pl.when(cid == 1)
  def _():
    pltpu.sync_copy(z0_hbm.at[rsl], agg.at[rsl])

  csl = pl.ds(wid * NCH_A, NCH_A)
  pltpu.sync_copy(src_hbm.at[csl], idx_s)
  pltpu.sync_copy(dst_hbm.at[csl], idx_d)

  plsc.subcore_barrier()

  rows = [rows0, rows1, rows2, rows3]
  sg = [sg0, sg1, sg2, sg3]
  ss = [ss0, ss1, ss2, ss3]

  def gissue(c, b):
    pltpu.async_copy(h_hbm.at[idx_s.at[c]], rows[b], sg[b])

  def consume(c, b):
    pltpu.make_async_copy(h_hbm.at[pl.ds(0, CH_A)], rows[b], sg[b]).wait()
    pltpu.async_copy(rows[b], agg.at[idx_d.at[c]], ss[b], add=True)

  def cdrain(b):
    pltpu.make_async_copy(h_hbm.at[pl.ds(0, CH_A)], rows[b], ss[b]).wait()

  _ring4(NCH_A, gissue, consume, cdrain)

  plsc.subcore_barrier()

  @pl.when(cid == 0)
  def _():
    pltpu.sync_copy(agg.at[rsl], out0_hbm.at[rsl])

  @pl.when(cid == 1)
  def _():
    pltpu.sync_copy(agg.at[rsl], out1_hbm.at[rsl])


_agg_call = pl.kernel(
    _agg_body,
    out_type=(jax.ShapeDtypeStruct((N, D), jnp.float32),
              jax.ShapeDtypeStruct((N, D), jnp.float32)),
    mesh=_MESH,
    compiler_params=pltpu.CompilerParams(use_tc_tiling_on_sc=False),
    scratch_types=(
        [pltpu.VMEM((NCH_A, CH_A), jnp.int32)] * 2
        + [pltpu.VMEM((CH_A, D), jnp.float32)] * 4
        + [pltpu.VMEM_SHARED((N, D), jnp.float32)]
        + [pltpu.SemaphoreType.DMA] * 8
    ),
)


ROWS_B = 1000


def _mlp_body(a0_ref, a1_ref, wa_ref, g_ref, c_ref, wb_ref, bb_ref, out_ref):
  z = a0_ref[...] + a1_ref[...]
  t = jnp.dot(z, wa_ref[...], preferred_element_type=jnp.float32)
  t = jnp.maximum(t * g_ref[...] + c_ref[...], 0.0)
  o = jnp.dot(t, wb_ref[...], preferred_element_type=jnp.float32)
  out_ref[...] = jnp.maximum(o + bb_ref[...], 0.0)


def _mlp_call(a0, a1, wa, g, c, wb, bb):
  grid = (N // ROWS_B,)
  row_spec = pl.BlockSpec((ROWS_B, D), lambda i: (i, 0))
  full = lambda shape: pl.BlockSpec(shape, lambda i: (0,) * len(shape))
  return pl.pallas_call(
      _mlp_body,
      grid=grid,
      in_specs=[row_spec, row_spec, full((D, D)), full((1, D)), full((1, D)),
                full((D, D)), full((1, D))],
      out_specs=row_spec,
      out_shape=jax.ShapeDtypeStruct((N, D), jnp.float32),
  )(a0, a1, wa, g.reshape(1, D), c.reshape(1, D), wb, bb.reshape(1, D))


def _head_body(h_ref, wl1_ref, bl1_ref, h2_ref):
  t = jnp.dot(h_ref[...], wl1_ref[...], preferred_element_type=jnp.float32)
  h2_ref[...] = jnp.maximum(t + bl1_ref[...], 0.0)


def _head_call(h, wl1, bl1):
  grid = (N // ROWS_B,)
  full = lambda shape: pl.BlockSpec(shape, lambda i: (0,) * len(shape))
  return pl.pallas_call(
      _head_body,
      grid=grid,
      in_specs=[pl.BlockSpec((ROWS_B, D), lambda i: (i, 0)),
                full((D, D2)), full((1, D2))],
      out_specs=pl.BlockSpec((ROWS_B, D2), lambda i: (i, 0)),
      out_shape=jax.ShapeDtypeStruct((N, D2), jnp.float32),
  )(h, wl1, bl1.reshape(1, D2))


MM_B = 2000


def _mm_body(e_ref, w_ref, b_ref, out_ref):
  out_ref[...] = jnp.dot(e_ref[...], w_ref[...],
                         preferred_element_type=jnp.float32) + b_ref[...]


def _mm_call(e, wl2, bl2):
  grid = (E // MM_B,)
  full = lambda shape: pl.BlockSpec(shape, lambda i: (0,) * len(shape))
  return pl.pallas_call(
      _mm_body,
      grid=grid,
      in_specs=[pl.BlockSpec((MM_B, D), lambda i: (i, 0)),
                full((D, NCLS)), full((1, NCLS))],
      out_specs=pl.BlockSpec((MM_B, NCLS), lambda i: (i, 0)),
      out_shape=jax.ShapeDtypeStruct((E, NCLS), jnp.float32),
  )(e, wl2, bl2.reshape(1, NCLS))


def _e_body(h2_hbm, src_hbm, dst_hbm, e_hbm,
            idx_s, idx_d,
            bs0, bs1, bs2, bs3, bd0, bd1, bd2, bd3,
            sg0, sg1, sg2, sg3, sw0, sw1, sw2, sw3):
  cid = lax.axis_index("c")
  sid = lax.axis_index("s")
  wid = sid * NC + cid

  csl = pl.ds(wid * NCH_E, NCH_E)
  pltpu.sync_copy(src_hbm.at[csl], idx_s)
  pltpu.sync_copy(dst_hbm.at[csl], idx_d)

  bs = [bs0, bs1, bs2, bs3]
  bd = [bd0, bd1, bd2, bd3]
  sg = [sg0, sg1, sg2, sg3]
  sw = [sw0, sw1, sw2, sw3]

  def gissue(c, b):
    pltpu.async_copy(h2_hbm.at[idx_s.at[c]], bs[b], sg[b])
    pltpu.async_copy(h2_hbm.at[idx_d.at[c]], bd[b], sg[b])

  def consume(c, b):
    pltpu.make_async_copy(h2_hbm.at[pl.ds(0, CH_E)], bs[b], sg[b]).wait()
    pltpu.make_async_copy(h2_hbm.at[pl.ds(0, CH_E)], bd[b], sg[b]).wait()
    esl = pl.ds((wid * NCH_E + c) * CH_E, CH_E)
    pltpu.async_copy(bs[b], e_hbm.at[esl, pl.ds(0, D2)], sw[b])
    pltpu.async_copy(bd[b], e_hbm.at[esl, pl.ds(D2, D2)], sw[b])

  def cdrain(b):
    pltpu.make_async_copy(h2_hbm.at[pl.ds(0, CH_E)], bs[b], sw[b]).wait()
    pltpu.make_async_copy(h2_hbm.at[pl.ds(0, CH_E)], bd[b], sw[b]).wait()

  _ring4(NCH_E, gissue, consume, cdrain)


_e_call = pl.kernel(
    _e_body,
    out_type=jax.ShapeDtypeStruct((E, D), jnp.float32),
    mesh=_MESH,
    compiler_params=pltpu.CompilerParams(use_tc_tiling_on_sc=False),
    scratch_types=(
        [pltpu.VMEM((NCH_E, CH_E), jnp.int32)] * 2
        + [pltpu.VMEM((CH_E, D2), jnp.float32)] * 8
        + [pltpu.SemaphoreType.DMA] * 8
    ),
)


@jax.jit
def kernel(x, edge_index, batch, Wa, ba, bng, bnb, Wb, bb, Wl1, bl1, Wl2, bl2):
  del batch
  src_a = edge_index[0].reshape(E // CH_A, CH_A)
  dst_a = edge_index[1].reshape(E // CH_A, CH_A)
  src_e = edge_index[0].reshape(E // CH_E, CH_E)
  dst_e = edge_index[1].reshape(E // CH_E, CH_E)
  z0 = jnp.zeros((N, D), jnp.float32)

  inv_std = 1.0 / math.sqrt(1.0 + 1e-5)
  g = bng * inv_std
  cterm = g * ba + bnb

  h = x
  for l in range(Wa.shape[0]):
    a0, a1 = _agg_call(h, z0, src_a, dst_a)
    h = _mlp_call(a0, a1, Wa[l], g[l], cterm[l], Wb[l], bb[l])

  h2 = _head_call(h, Wl1, bl1)
  e = _e_call(h2, src_e, dst_e)
  out = _mm_call(e, Wl2, bl2)
  return (out, e)


# speedup vs baseline: 1.0882x; 1.0882x over previous
import functools
import math

import jax
import jax.numpy as jnp
from jax import lax
from jax.experimental import pallas as pl
from jax.experimental.pallas import tpu as pltpu
from jax.experimental.pallas import tpu_sc as plsc

N = 10000
E = 320000
D = 128
D2 = 64
NCLS = 10
NCLS_PAD = 16

NC = 2
NS = 16
NW = NC * NS
EPW = E // NW

CH_A = 40
NCH_A = EPW // CH_A
CH_E = 80
NCH_E = EPW // CH_E

ROWS_PER_TILE = N // NS

_MESH = plsc.VectorSubcoreMesh(core_axis_name="c", subcore_axis_name="s")


def _ring4(nchunk, gissue, consume, cdrain):
  gissue(0, 0)
  gissue(1, 1)

  @pl.loop(0, nchunk // 4)
  def _(j):
    for p in range(4):
      c = 4 * j + p
      b2 = (p + 2) % 4
      if p >= 2:
        cdrain(b2)
      else:
        pl.when(c >= 2)(functools.partial(cdrain, b2))
      pl.when(c + 2 < nchunk)(functools.partial(gissue, c + 2, b2))
      consume(c, p)

  rem = nchunk % 4
  for t in range(rem):
    consume(nchunk - rem + t, t)
  for c in range(nchunk - rem - 2, nchunk):
    cdrain(c % 4)


def _agg_body(h_hbm, z0_hbm, src_hbm, dst_hbm, out0_hbm, out1_hbm,
              idx_s, idx_d, rows0, rows1, rows2, rows3, agg,
              sg0, sg1, sg2, sg3, ss0, ss1, ss2, ss3):
  cid = lax.axis_index("c")
  sid = lax.axis_index("s")
  wid = sid * NC + cid

  rsl = pl.ds(sid * ROWS_PER_TILE, ROWS_PER_TILE)

  @pl.when(cid == 0)
  def _():
    pltpu.sync_copy(h_hbm.at[rsl], agg.at[rsl])

  @pl.when(cid == 1)
  def _():
    pltpu.sync_copy(z0_hbm.at[rsl], agg.at[rsl])

  csl = pl.ds(wid * NCH_A, NCH_A)
  pltpu.sync_copy(src_hbm.at[csl], idx_s)
  pltpu.sync_copy(dst_hbm.at[csl], idx_d)

  plsc.subcore_barrier()

  rows = [rows0, rows1, rows2, rows3]
  sg = [sg0, sg1, sg2, sg3]
  ss = [ss0, ss1, ss2, ss3]

  def gissue(c, b):
    pltpu.async_copy(h_hbm.at[idx_s.at[c]], rows[b], sg[b])

  def consume(c, b):
    pltpu.make_async_copy(h_hbm.at[pl.ds(0, CH_A)], rows[b], sg[b]).wait()
    pltpu.async_copy(rows[b], agg.at[idx_d.at[c]], ss[b], add=True)

  def cdrain(b):
    pltpu.make_async_copy(h_hbm.at[pl.ds(0, CH_A)], rows[b], ss[b]).wait()

  _ring4(NCH_A, gissue, consume, cdrain)

  plsc.subcore_barrier()

  @pl.when(cid == 0)
  def _():
    pltpu.sync_copy(agg.at[rsl], out0_hbm.at[rsl])

  @pl.when(cid == 1)
  def _():
    pltpu.sync_copy(agg.at[rsl], out1_hbm.at[rsl])


_agg_call = pl.kernel(
    _agg_body,
    out_type=(jax.ShapeDtypeStruct((N, D), jnp.float32),
              jax.ShapeDtypeStruct((N, D), jnp.float32)),
    mesh=_MESH,
    compiler_params=pltpu.CompilerParams(use_tc_tiling_on_sc=False),
    scratch_types=(
        [pltpu.VMEM((NCH_A, CH_A), jnp.int32)] * 2
        + [pltpu.VMEM((CH_A, D), jnp.float32)] * 4
        + [pltpu.VMEM_SHARED((N, D), jnp.float32)]
        + [pltpu.SemaphoreType.DMA] * 8
    ),
)


ROWS_B = 1000


def _mlp_body(a0_ref, a1_ref, wa_ref, g_ref, c_ref, wb_ref, bb_ref, out_ref):
  z = a0_ref[...] + a1_ref[...]
  t = jnp.dot(z, wa_ref[...], preferred_element_type=jnp.float32)
  t = jnp.maximum(t * g_ref[...] + c_ref[...], 0.0)
  o = jnp.dot(t, wb_ref[...], preferred_element_type=jnp.float32)
  out_ref[...] = jnp.maximum(o + bb_ref[...], 0.0)


def _mlp_call(a0, a1, wa, g, c, wb, bb):
  grid = (N // ROWS_B,)
  row_spec = pl.BlockSpec((ROWS_B, D), lambda i: (i, 0))
  full = lambda shape: pl.BlockSpec(shape, lambda i: (0,) * len(shape))
  return pl.pallas_call(
      _mlp_body,
      grid=grid,
      in_specs=[row_spec, row_spec, full((D, D)), full((1, D)), full((1, D)),
                full((D, D)), full((1, D))],
      out_specs=row_spec,
      out_shape=jax.ShapeDtypeStruct((N, D), jnp.float32),
  )(a0, a1, wa, g.reshape(1, D), c.reshape(1, D), wb, bb.reshape(1, D))


def _head_body(h_ref, wl1_ref, bl1_ref, w2a_ref, w2b_ref, bl2_ref,
               h2_ref, p_ref, q_ref):
  t = jnp.dot(h_ref[...], wl1_ref[...], preferred_element_type=jnp.float32)
  h2 = jnp.maximum(t + bl1_ref[...], 0.0)
  h2_ref[...] = h2
  p_ref[...] = jnp.dot(h2, w2a_ref[...], preferred_element_type=jnp.float32) + bl2_ref[...]
  q_ref[...] = jnp.dot(h2, w2b_ref[...], preferred_element_type=jnp.float32)


def _head_call(h, wl1, bl1, w2a, w2b, bl2p):
  grid = (N // ROWS_B,)
  full = lambda shape: pl.BlockSpec(shape, lambda i: (0,) * len(shape))
  return pl.pallas_call(
      _head_body,
      grid=grid,
      in_specs=[pl.BlockSpec((ROWS_B, D), lambda i: (i, 0)),
                full((D, D2)), full((1, D2)),
                full((D2, NCLS_PAD)), full((D2, NCLS_PAD)), full((1, NCLS_PAD))],
      out_specs=[pl.BlockSpec((ROWS_B, D2), lambda i: (i, 0)),
                 pl.BlockSpec((ROWS_B, NCLS_PAD), lambda i: (i, 0)),
                 pl.BlockSpec((ROWS_B, NCLS_PAD), lambda i: (i, 0))],
      out_shape=[jax.ShapeDtypeStruct((N, D2), jnp.float32),
                 jax.ShapeDtypeStruct((N, NCLS_PAD), jnp.float32),
                 jax.ShapeDtypeStruct((N, NCLS_PAD), jnp.float32)],
  )(h, wl1, bl1.reshape(1, D2), w2a, w2b, bl2p.reshape(1, NCLS_PAD))


def _out_body(p_hbm, q_hbm, src_hbm, dst_hbm, out_hbm,
              idx_s, idx_d,
              ps0, ps1, ps2, ps3, qd0, qd1, qd2, qd3,
              sg0, sg1, sg2, sg3, sw0, sw1, sw2, sw3):
  cid = lax.axis_index("c")
  sid = lax.axis_index("s")
  wid = sid * NC + cid

  csl = pl.ds(wid * NCH_E, NCH_E)
  pltpu.sync_copy(src_hbm.at[csl], idx_s)
  pltpu.sync_copy(dst_hbm.at[csl], idx_d)

  ps = [ps0, ps1, ps2, ps3]
  qd = [qd0, qd1, qd2, qd3]
  sg = [sg0, sg1, sg2, sg3]
  sw = [sw0, sw1, sw2, sw3]

  def gissue(c, b):
    pltpu.async_copy(p_hbm.at[idx_s.at[c]], ps[b], sg[b])
    pltpu.async_copy(q_hbm.at[idx_d.at[c]], qd[b], sg[b])

  def consume(c, b):
    pltpu.make_async_copy(p_hbm.at[pl.ds(0, CH_E)], ps[b], sg[b]).wait()
    pltpu.make_async_copy(q_hbm.at[pl.ds(0, CH_E)], qd[b], sg[b]).wait()

    @pl.loop(0, CH_E, unroll=8)
    def _(i):
      ps[b][i, :] = ps[b][i, :] + qd[b][i, :]

    esl = pl.ds((wid * NCH_E + c) * CH_E, CH_E)
    pltpu.async_copy(ps[b], out_hbm.at[esl], sw[b])

  def cdrain(b):
    pltpu.make_async_copy(p_hbm.at[pl.ds(0, CH_E)], ps[b], sw[b]).wait()

  _ring4(NCH_E, gissue, consume, cdrain)


_out_call = pl.kernel(
    _out_body,
    out_type=jax.ShapeDtypeStruct((E, NCLS_PAD), jnp.float32),
    mesh=_MESH,
    compiler_params=pltpu.CompilerParams(use_tc_tiling_on_sc=False),
    scratch_types=(
        [pltpu.VMEM((NCH_E, CH_E), jnp.int32)] * 2
        + [pltpu.VMEM((CH_E, NCLS_PAD), jnp.float32)] * 8
        + [pltpu.SemaphoreType.DMA] * 8
    ),
)


def _e_body(h2_hbm, src_hbm, dst_hbm, e_hbm,
            idx_s, idx_d,
            bs0, bs1, bs2, bs3, bd0, bd1, bd2, bd3,
            sg0, sg1, sg2, sg3, sw0, sw1, sw2, sw3):
  cid = lax.axis_index("c")
  sid = lax.axis_index("s")
  wid = sid * NC + cid

  csl = pl.ds(wid * NCH_E, NCH_E)
  pltpu.sync_copy(src_hbm.at[csl], idx_s)
  pltpu.sync_copy(dst_hbm.at[csl], idx_d)

  bs = [bs0, bs1, bs2, bs3]
  bd = [bd0, bd1, bd2, bd3]
  sg = [sg0, sg1, sg2, sg3]
  sw = [sw0, sw1, sw2, sw3]

  def gissue(c, b):
    pltpu.async_copy(h2_hbm.at[idx_s.at[c]], bs[b], sg[b])
    pltpu.async_copy(h2_hbm.at[idx_d.at[c]], bd[b], sg[b])

  def consume(c, b):
    pltpu.make_async_copy(h2_hbm.at[pl.ds(0, CH_E)], bs[b], sg[b]).wait()
    pltpu.make_async_copy(h2_hbm.at[pl.ds(0, CH_E)], bd[b], sg[b]).wait()
    esl = pl.ds((wid * NCH_E + c) * CH_E, CH_E)
    pltpu.async_copy(bs[b], e_hbm.at[esl, pl.ds(0, D2)], sw[b])
    pltpu.async_copy(bd[b], e_hbm.at[esl, pl.ds(D2, D2)], sw[b])

  def cdrain(b):
    pltpu.make_async_copy(h2_hbm.at[pl.ds(0, CH_E)], bs[b], sw[b]).wait()
    pltpu.make_async_copy(h2_hbm.at[pl.ds(0, CH_E)], bd[b], sw[b]).wait()

  _ring4(NCH_E, gissue, consume, cdrain)


_e_call = pl.kernel(
    _e_body,
    out_type=jax.ShapeDtypeStruct((E, D), jnp.float32),
    mesh=_MESH,
    compiler_params=pltpu.CompilerParams(use_tc_tiling_on_sc=False),
    scratch_types=(
        [pltpu.VMEM((NCH_E, CH_E), jnp.int32)] * 2
        + [pltpu.VMEM((CH_E, D2), jnp.float32)] * 8
        + [pltpu.SemaphoreType.DMA] * 8
    ),
)


@jax.jit
def kernel(x, edge_index, batch, Wa, ba, bng, bnb, Wb, bb, Wl1, bl1, Wl2, bl2):
  del batch
  src_a = edge_index[0].reshape(E // CH_A, CH_A)
  dst_a = edge_index[1].reshape(E // CH_A, CH_A)
  src_e = edge_index[0].reshape(E // CH_E, CH_E)
  dst_e = edge_index[1].reshape(E // CH_E, CH_E)
  z0 = jnp.zeros((N, D), jnp.float32)

  inv_std = 1.0 / math.sqrt(1.0 + 1e-5)
  g = bng * inv_std
  cterm = g * ba + bnb

  h = x
  for l in range(Wa.shape[0]):
    a0, a1 = _agg_call(h, z0, src_a, dst_a)
    h = _mlp_call(a0, a1, Wa[l], g[l], cterm[l], Wb[l], bb[l])

  w2a = jnp.pad(Wl2[:D2], ((0, 0), (0, NCLS_PAD - NCLS)))
  w2b = jnp.pad(Wl2[D2:], ((0, 0), (0, NCLS_PAD - NCLS)))
  bl2p = jnp.pad(bl2, (0, NCLS_PAD - NCLS))
  h2, p, q = _head_call(h, Wl1, bl1, w2a, w2b, bl2p)

  outp = _out_call(p, q, src_e, dst_e)
  e = _e_call(h2, src_e, dst_e)
  return (outp[:, :NCLS], e)


# speedup vs baseline: 1.0897x; 1.0013x over previous
import functools
import math

import jax
import jax.numpy as jnp
from jax import lax
from jax.experimental import pallas as pl
from jax.experimental.pallas import tpu as pltpu
from jax.experimental.pallas import tpu_sc as plsc

N = 10000
E = 320000
D = 128
D2 = 64
NCLS = 10
NCLS_PAD = 16

NC = 2
NS = 16
NW = NC * NS
EPW = E // NW

CH_A = 40
NCH_A = EPW // CH_A
CH_E = 80
NCH_E = EPW // CH_E

ROWS_PER_TILE = N // NS

_MESH = plsc.VectorSubcoreMesh(core_axis_name="c", subcore_axis_name="s")


def _ring4(nchunk, gissue, consume, cdrain):
  gissue(0, 0)
  gissue(1, 1)

  @pl.loop(0, nchunk // 4)
  def _(j):
    for p in range(4):
      c = 4 * j + p
      b2 = (p + 2) % 4
      if p >= 2:
        cdrain(b2)
      else:
        pl.when(c >= 2)(functools.partial(cdrain, b2))
      pl.when(c + 2 < nchunk)(functools.partial(gissue, c + 2, b2))
      consume(c, p)

  rem = nchunk % 4
  for t in range(rem):
    consume(nchunk - rem + t, t)
  for c in range(nchunk - rem - 2, nchunk):
    cdrain(c % 4)


def _agg_body(h_hbm, z0_hbm, src_hbm, dst_hbm, out0_hbm, out1_hbm,
              idx_s, idx_d, rows0, rows1, rows2, rows3, agg,
              sg0, sg1, sg2, sg3, ss0, ss1, ss2, ss3):
  cid = lax.axis_index("c")
  sid = lax.axis_index("s")
  wid = sid * NC + cid

  rsl = pl.ds(sid * ROWS_PER_TILE, ROWS_PER_TILE)

  @pl.when(cid == 0)
  def _():
    pltpu.sync_copy(h_hbm.at[rsl], agg.at[rsl])

  @pl.when(cid == 1)
  def _():
    pltpu.sync_copy(z0_hbm.at[rsl], agg.at[rsl])

  csl = pl.ds(wid * EPW, EPW)
  pltpu.sync_copy(src_hbm.at[csl], idx_s)
  pltpu.sync_copy(dst_hbm.at[csl], idx_d)

  plsc.subcore_barrier()

  rows = [rows0, rows1, rows2, rows3]
  sg = [sg0, sg1, sg2, sg3]
  ss = [ss0, ss1, ss2, ss3]

  def gissue(c, b):
    pltpu.async_copy(h_hbm.at[idx_s.at[pl.ds(c * CH_A, CH_A)]], rows[b], sg[b])

  def consume(c, b):
    pltpu.make_async_copy(h_hbm.at[pl.ds(0, CH_A)], rows[b], sg[b]).wait()
    pltpu.async_copy(rows[b], agg.at[idx_d.at[pl.ds(c * CH_A, CH_A)]],
                     ss[b], add=True)

  def cdrain(b):
    pltpu.make_async_copy(h_hbm.at[pl.ds(0, CH_A)], rows[b], ss[b]).wait()

  _ring4(NCH_A, gissue, consume, cdrain)

  plsc.subcore_barrier()

  @pl.when(cid == 0)
  def _():
    pltpu.sync_copy(agg.at[rsl], out0_hbm.at[rsl])

  @pl.when(cid == 1)
  def _():
    pltpu.sync_copy(agg.at[rsl], out1_hbm.at[rsl])


_agg_call = pl.kernel(
    _agg_body,
    out_type=(jax.ShapeDtypeStruct((N, D), jnp.float32),
              jax.ShapeDtypeStruct((N, D), jnp.float32)),
    mesh=_MESH,
    compiler_params=pltpu.CompilerParams(use_tc_tiling_on_sc=False),
    scratch_types=(
        [pltpu.VMEM((EPW,), jnp.int32)] * 2
        + [pltpu.VMEM((CH_A, D), jnp.float32)] * 4
        + [pltpu.VMEM_SHARED((N, D), jnp.float32)]
        + [pltpu.SemaphoreType.DMA] * 8
    ),
)


ROWS_B = 1000


def _mlp_body(a0_ref, a1_ref, wa_ref, g_ref, c_ref, wb_ref, bb_ref, out_ref):
  z = a0_ref[...] + a1_ref[...]
  t = jnp.dot(z, wa_ref[...], preferred_element_type=jnp.float32)
  t = jnp.maximum(t * g_ref[...] + c_ref[...], 0.0)
  o = jnp.dot(t, wb_ref[...], preferred_element_type=jnp.float32)
  out_ref[...] = jnp.maximum(o + bb_ref[...], 0.0)


def _mlp_call(a0, a1, wa, g, c, wb, bb):
  grid = (N // ROWS_B,)
  row_spec = pl.BlockSpec((ROWS_B, D), lambda i: (i, 0))
  full = lambda shape: pl.BlockSpec(shape, lambda i: (0,) * len(shape))
  return pl.pallas_call(
      _mlp_body,
      grid=grid,
      in_specs=[row_spec, row_spec, full((D, D)), full((1, D)), full((1, D)),
                full((D, D)), full((1, D))],
      out_specs=row_spec,
      out_shape=jax.ShapeDtypeStruct((N, D), jnp.float32),
  )(a0, a1, wa, g.reshape(1, D), c.reshape(1, D), wb, bb.reshape(1, D))


def _head_body(h_ref, wl1_ref, bl1_ref, w2a_ref, w2b_ref, bl2_ref,
               h2_ref, p_ref, q_ref):
  t = jnp.dot(h_ref[...], wl1_ref[...], preferred_element_type=jnp.float32)
  h2 = jnp.maximum(t + bl1_ref[...], 0.0)
  h2_ref[...] = h2
  p_ref[...] = jnp.dot(h2, w2a_ref[...], preferred_element_type=jnp.float32) + bl2_ref[...]
  q_ref[...] = jnp.dot(h2, w2b_ref[...], preferred_element_type=jnp.float32)


def _head_call(h, wl1, bl1, w2a, w2b, bl2p):
  grid = (N // ROWS_B,)
  full = lambda shape: pl.BlockSpec(shape, lambda i: (0,) * len(shape))
  return pl.pallas_call(
      _head_body,
      grid=grid,
      in_specs=[pl.BlockSpec((ROWS_B, D), lambda i: (i, 0)),
                full((D, D2)), full((1, D2)),
                full((D2, NCLS_PAD)), full((D2, NCLS_PAD)), full((1, NCLS_PAD))],
      out_specs=[pl.BlockSpec((ROWS_B, D2), lambda i: (i, 0)),
                 pl.BlockSpec((ROWS_B, NCLS_PAD), lambda i: (i, 0)),
                 pl.BlockSpec((ROWS_B, NCLS_PAD), lambda i: (i, 0))],
      out_shape=[jax.ShapeDtypeStruct((N, D2), jnp.float32),
                 jax.ShapeDtypeStruct((N, NCLS_PAD), jnp.float32),
                 jax.ShapeDtypeStruct((N, NCLS_PAD), jnp.float32)],
  )(h, wl1, bl1.reshape(1, D2), w2a, w2b, bl2p.reshape(1, NCLS_PAD))


def _out_body(p_hbm, q_hbm, src_hbm, dst_hbm, out_hbm,
              idx_s, idx_d,
              ps0, ps1, ps2, ps3, qd0, qd1, qd2, qd3,
              sg0, sg1, sg2, sg3, sw0, sw1, sw2, sw3):
  cid = lax.axis_index("c")
  sid = lax.axis_index("s")
  wid = sid * NC + cid

  csl = pl.ds(wid * EPW, EPW)
  pltpu.sync_copy(src_hbm.at[csl], idx_s)
  pltpu.sync_copy(dst_hbm.at[csl], idx_d)

  ps = [ps0, ps1, ps2, ps3]
  qd = [qd0, qd1, qd2, qd3]
  sg = [sg0, sg1, sg2, sg3]
  sw = [sw0, sw1, sw2, sw3]

  def gissue(c, b):
    esl = pl.ds(c * CH_E, CH_E)
    pltpu.async_copy(p_hbm.at[idx_s.at[esl]], ps[b], sg[b])
    pltpu.async_copy(q_hbm.at[idx_d.at[esl]], qd[b], sg[b])

  def consume(c, b):
    pltpu.make_async_copy(p_hbm.at[pl.ds(0, CH_E)], ps[b], sg[b]).wait()
    pltpu.make_async_copy(q_hbm.at[pl.ds(0, CH_E)], qd[b], sg[b]).wait()

    @pl.loop(0, CH_E, unroll=8)
    def _(i):
      ps[b][i, :] = ps[b][i, :] + qd[b][i, :]

    esl = pl.ds((wid * NCH_E + c) * CH_E, CH_E)
    pltpu.async_copy(ps[b], out_hbm.at[esl], sw[b])

  def cdrain(b):
    pltpu.make_async_copy(p_hbm.at[pl.ds(0, CH_E)], ps[b], sw[b]).wait()

  _ring4(NCH_E, gissue, consume, cdrain)


_out_call = pl.kernel(
    _out_body,
    out_type=jax.ShapeDtypeStruct((E, NCLS_PAD), jnp.float32),
    mesh=_MESH,
    compiler_params=pltpu.CompilerParams(use_tc_tiling_on_sc=False),
    scratch_types=(
        [pltpu.VMEM((EPW,), jnp.int32)] * 2
        + [pltpu.VMEM((CH_E, NCLS_PAD), jnp.float32)] * 8
        + [pltpu.SemaphoreType.DMA] * 8
    ),
)


def _e_body(h2_hbm, src_hbm, dst_hbm, e_hbm,
            idx_s, idx_d,
            bs0, bs1, bs2, bs3, bd0, bd1, bd2, bd3,
            sg0, sg1, sg2, sg3, sw0, sw1, sw2, sw3):
  cid = lax.axis_index("c")
  sid = lax.axis_index("s")
  wid = sid * NC + cid

  csl = pl.ds(wid * EPW, EPW)
  pltpu.sync_copy(src_hbm.at[csl], idx_s)
  pltpu.sync_copy(dst_hbm.at[csl], idx_d)

  bs = [bs0, bs1, bs2, bs3]
  bd = [bd0, bd1, bd2, bd3]
  sg = [sg0, sg1, sg2, sg3]
  sw = [sw0, sw1, sw2, sw3]

  def gissue(c, b):
    isl = pl.ds(c * CH_E, CH_E)
    pltpu.async_copy(h2_hbm.at[idx_s.at[isl]], bs[b], sg[b])
    pltpu.async_copy(h2_hbm.at[idx_d.at[isl]], bd[b], sg[b])

  def consume(c, b):
    pltpu.make_async_copy(h2_hbm.at[pl.ds(0, CH_E)], bs[b], sg[b]).wait()
    pltpu.make_async_copy(h2_hbm.at[pl.ds(0, CH_E)], bd[b], sg[b]).wait()
    esl = pl.ds((wid * NCH_E + c) * CH_E, CH_E)
    pltpu.async_copy(bs[b], e_hbm.at[esl, pl.ds(0, D2)], sw[b])
    pltpu.async_copy(bd[b], e_hbm.at[esl, pl.ds(D2, D2)], sw[b])

  def cdrain(b):
    pltpu.make_async_copy(h2_hbm.at[pl.ds(0, CH_E)], bs[b], sw[b]).wait()
    pltpu.make_async_copy(h2_hbm.at[pl.ds(0, CH_E)], bd[b], sw[b]).wait()

  _ring4(NCH_E, gissue, consume, cdrain)


_e_call = pl.kernel(
    _e_body,
    out_type=jax.ShapeDtypeStruct((E, D), jnp.float32),
    mesh=_MESH,
    compiler_params=pltpu.CompilerParams(use_tc_tiling_on_sc=False),
    scratch_types=(
        [pltpu.VMEM((EPW,), jnp.int32)] * 2
        + [pltpu.VMEM((CH_E, D2), jnp.float32)] * 8
        + [pltpu.SemaphoreType.DMA] * 8
    ),
)


@jax.jit
def kernel(x, edge_index, batch, Wa, ba, bng, bnb, Wb, bb, Wl1, bl1, Wl2, bl2):
  del batch
  src = edge_index[0]
  dst = edge_index[1]
  z0 = jnp.zeros((N, D), jnp.float32)

  inv_std = 1.0 / math.sqrt(1.0 + 1e-5)
  g = bng * inv_std
  cterm = g * ba + bnb

  h = x
  for l in range(Wa.shape[0]):
    a0, a1 = _agg_call(h, z0, src, dst)
    h = _mlp_call(a0, a1, Wa[l], g[l], cterm[l], Wb[l], bb[l])

  w2a = jnp.pad(Wl2[:D2], ((0, 0), (0, NCLS_PAD - NCLS)))
  w2b = jnp.pad(Wl2[D2:], ((0, 0), (0, NCLS_PAD - NCLS)))
  bl2p = jnp.pad(bl2, (0, NCLS_PAD - NCLS))
  h2, p, q = _head_call(h, Wl1, bl1, w2a, w2b, bl2p)

  outp = _out_call(p, q, src, dst)
  e = _e_call(h2, src, dst)
  return (outp[:, :NCLS], e)
